# baseline (speedup 1.0000x reference)
import jax
import jax.numpy as jnp
from jax.experimental import pallas as pl
from jax.experimental.pallas import tpu as pltpu

_TOTAL_TOK = 16384
_OLD_DIM = 256
_HID_DIM = 512
_NEW_DIM = 128
_TILE = 8192


def _mlp_tile(x_ref, w1_ref, b1_ref, w2_ref, b2_ref, o_ref):
    x = x_ref[...].astype(jnp.bfloat16)
    w1 = w1_ref[...].astype(jnp.bfloat16)
    h = jax.lax.dot_general(
        x, w1, (((1,), (0,)), ((), ())), preferred_element_type=jnp.float32
    )
    h = jnp.maximum(
        h.astype(jnp.bfloat16) + b1_ref[...].astype(jnp.bfloat16),
        jnp.bfloat16(0.0),
    )
    w2 = w2_ref[...].astype(jnp.bfloat16)
    o = jax.lax.dot_general(
        h, w2, (((1,), (0,)), ((), ())), preferred_element_type=jnp.float32
    )
    o_ref[...] = o + b2_ref[...]


def kernel(flat, cu_seqlens, W1, b1, W2, b2):
    del cu_seqlens
    b1r = jnp.reshape(b1, (1, _HID_DIM))
    b2r = jnp.reshape(b2, (1, _NEW_DIM))
    grid = (_TOTAL_TOK // _TILE,)
    out = pl.pallas_call(
        _mlp_tile,
        grid=grid,
        in_specs=[
            pl.BlockSpec((_TILE, _OLD_DIM), lambda i: (i, 0)),
            pl.BlockSpec((_OLD_DIM, _HID_DIM), lambda i: (0, 0)),
            pl.BlockSpec((1, _HID_DIM), lambda i: (0, 0)),
            pl.BlockSpec((_HID_DIM, _NEW_DIM), lambda i: (0, 0)),
            pl.BlockSpec((1, _NEW_DIM), lambda i: (0, 0)),
        ],
        out_specs=pl.BlockSpec((_TILE, _NEW_DIM), lambda i: (i, 0)),
        out_shape=jax.ShapeDtypeStruct((_TOTAL_TOK, _NEW_DIM), jnp.float32),
        compiler_params=pltpu.CompilerParams(
            dimension_semantics=("arbitrary",),
        ),
    )(flat, W1, b1r, W2, b2r)
    return out


# speedup vs baseline: 1.0557x; 1.0557x over previous
import jax
import jax.numpy as jnp
from jax.experimental import pallas as pl
from jax.experimental.pallas import tpu as pltpu

_TOTAL_TOK = 16384
_OLD_DIM = 256
_HID_DIM = 512
_NEW_DIM = 128
_TILE = 4096
_HALF = _TILE // 2


def _mlp_half(x, w1_ref, b1_ref, w2_ref, b2_ref):
    h = jax.lax.dot_general(
        x, w1_ref[...].astype(jnp.bfloat16), (((1,), (0,)), ((), ())),
        preferred_element_type=jnp.float32,
    )
    h = jnp.maximum(
        h.astype(jnp.bfloat16) + b1_ref[...].astype(jnp.bfloat16),
        jnp.bfloat16(0.0),
    )
    o = jax.lax.dot_general(
        h, w2_ref[...].astype(jnp.bfloat16), (((1,), (0,)), ((), ())),
        preferred_element_type=jnp.float32,
    )
    return o + b2_ref[...]


def _mlp_tile(x_ref, w1_ref, b1_ref, w2_ref, b2_ref, o_ref):
    xa = x_ref[:_HALF, :].astype(jnp.bfloat16)
    xb = x_ref[_HALF:, :].astype(jnp.bfloat16)
    o_ref[:_HALF, :] = _mlp_half(xa, w1_ref, b1_ref, w2_ref, b2_ref)
    o_ref[_HALF:, :] = _mlp_half(xb, w1_ref, b1_ref, w2_ref, b2_ref)


def kernel(flat, cu_seqlens, W1, b1, W2, b2):
    del cu_seqlens
    b1r = jnp.reshape(b1, (1, _HID_DIM))
    b2r = jnp.reshape(b2, (1, _NEW_DIM))
    grid = (_TOTAL_TOK // _TILE,)
    out = pl.pallas_call(
        _mlp_tile,
        grid=grid,
        in_specs=[
            pl.BlockSpec((_TILE, _OLD_DIM), lambda i: (i, 0)),
            pl.BlockSpec((_OLD_DIM, _HID_DIM), lambda i: (0, 0)),
            pl.BlockSpec((1, _HID_DIM), lambda i: (0, 0)),
            pl.BlockSpec((_HID_DIM, _NEW_DIM), lambda i: (0, 0)),
            pl.BlockSpec((1, _NEW_DIM), lambda i: (0, 0)),
        ],
        out_specs=pl.BlockSpec((_TILE, _NEW_DIM), lambda i: (i, 0)),
        out_shape=jax.ShapeDtypeStruct((_TOTAL_TOK, _NEW_DIM), jnp.float32),
        compiler_params=pltpu.CompilerParams(
            dimension_semantics=("parallel",),
        ),
    )(flat, W1, b1r, W2, b2r)
    return out
